# baseline (device time: 105620 ns/iter reference)
import numpy as np

import jax
import jax.numpy as jnp
from jax import lax
from jax.experimental import pallas as pl
from jax.experimental.pallas import tpu as pltpu

N_DEV = 32

_RING = np.array(
    [1, 2, 5, 6, 14, 13, 10, 9, 17, 18, 21, 22, 30, 29, 26, 25,
     24, 27, 28, 31, 23, 20, 19, 16, 8, 11, 12, 15, 7, 4, 3, 0],
    dtype=np.int32,
)
_POS = np.zeros(N_DEV, dtype=np.int32)
_POS[_RING] = np.arange(N_DEV, dtype=np.int32)

N_R = N_DEV // 2
N_L = N_DEV // 2 - 1
PIECES = 4


def kernel(x):
    m_per, n = x.shape

    def body(ring_ref, pos_ref, x_ref, out_ref,
             send_r, recv_r, send_l, recv_l):
        my = lax.axis_index("i")
        r = pos_ref[my]
        right = ring_ref[(r + 1) % N_DEV]
        left = ring_ref[(r + N_DEV - 1) % N_DEV]

        def chunk_at(ring_offset_back):
            return ring_ref[(r + N_DEV - ring_offset_back) % N_DEV]

        sub = m_per // PIECES

        def make(idx, piece, s_sem, r_sem, dev):
            sl = pl.ds(idx * m_per + piece * sub, sub)
            return pltpu.make_async_remote_copy(
                src_ref=out_ref.at[sl, :],
                dst_ref=out_ref.at[sl, :],
                send_sem=s_sem,
                recv_sem=r_sem,
                device_id=(dev,),
                device_id_type=pl.DeviceIdType.MESH,
            )

        barrier_sem = pltpu.get_barrier_semaphore()
        for nbr in (left, right):
            pl.semaphore_signal(
                barrier_sem,
                inc=1,
                device_id=(nbr,),
                device_id_type=pl.DeviceIdType.MESH,
            )
        pl.semaphore_wait(barrier_sem, 2)

        out_ref[pl.ds(my * m_per, m_per), :] = x_ref[:, :].astype(
            out_ref.dtype
        )

        sends = []
        for p in range(PIECES):
            s = make(my, p, send_r.at[p], recv_r.at[p], right)
            s.start()
            sends.append(s)
            s = make(my, p, send_l.at[p], recv_l.at[p], left)
            s.start()
            sends.append(s)

        for h in range(N_R):
            cr = chunk_at(h + 1)
            cl = chunk_at(N_DEV - h - 1)
            for p in range(PIECES):
                i = PIECES * h + p
                make(cr, p, send_r.at[i], recv_r.at[i], left).wait_recv()
                if h + 1 < N_R:
                    s = make(
                        cr, p,
                        send_r.at[i + PIECES], recv_r.at[i + PIECES], right,
                    )
                    s.start()
                    sends.append(s)
                if h < N_L:
                    make(cl, p, send_l.at[i], recv_l.at[i], right).wait_recv()
                    if h + 1 < N_L:
                        s = make(
                            cl, p,
                            send_l.at[i + PIECES], recv_l.at[i + PIECES], left,
                        )
                        s.start()
                        sends.append(s)

        for s in sends:
            s.wait_send()

    ring_tab = jnp.asarray(_RING)
    pos_tab = jnp.asarray(_POS)
    return pl.pallas_call(
        body,
        out_shape=jax.ShapeDtypeStruct((N_DEV * m_per, n), jnp.bfloat16),
        in_specs=[
            pl.BlockSpec(memory_space=pltpu.SMEM),
            pl.BlockSpec(memory_space=pltpu.SMEM),
            pl.BlockSpec(memory_space=pltpu.VMEM),
        ],
        out_specs=pl.BlockSpec(memory_space=pltpu.VMEM),
        scratch_shapes=[
            pltpu.SemaphoreType.DMA((PIECES * N_R,)),
            pltpu.SemaphoreType.DMA((PIECES * N_R,)),
            pltpu.SemaphoreType.DMA((PIECES * N_L,)),
            pltpu.SemaphoreType.DMA((PIECES * N_L,)),
        ],
        compiler_params=pltpu.CompilerParams(collective_id=0),
    )(ring_tab, pos_tab, x)


# device time: 103812 ns/iter; 1.0174x vs baseline; 1.0174x over previous
import numpy as np

import jax
import jax.numpy as jnp
from jax import lax
from jax.experimental import pallas as pl
from jax.experimental.pallas import tpu as pltpu

N_DEV = 32

_RING = np.array(
    [1, 2, 5, 6, 14, 13, 10, 9, 17, 18, 21, 22, 30, 29, 26, 25,
     24, 27, 28, 31, 23, 20, 19, 16, 8, 11, 12, 15, 7, 4, 3, 0],
    dtype=np.int32,
)
_POS = np.zeros(N_DEV, dtype=np.int32)
_POS[_RING] = np.arange(N_DEV, dtype=np.int32)

N_HOP = N_DEV // 2


def kernel(x):
    m_per, n = x.shape

    def body(ring_ref, pos_ref, x_ref, out_ref,
             send_r, recv_r, send_l, recv_l):
        my = lax.axis_index("i")
        r = pos_ref[my]
        right = ring_ref[(r + 1) % N_DEV]
        left = ring_ref[(r + N_DEV - 1) % N_DEV]

        def chunk_at(back):
            return ring_ref[(r + N_DEV - back) % N_DEV]

        half = m_per // 2

        def make(idx, piece, s_sem, r_sem, dev):
            sl = pl.ds(idx * m_per + piece * half, half)
            return pltpu.make_async_remote_copy(
                src_ref=out_ref.at[sl, :],
                dst_ref=out_ref.at[sl, :],
                send_sem=s_sem,
                recv_sem=r_sem,
                device_id=(dev,),
                device_id_type=pl.DeviceIdType.MESH,
            )

        barrier_sem = pltpu.get_barrier_semaphore()
        for nbr in (left, right):
            pl.semaphore_signal(
                barrier_sem,
                inc=1,
                device_id=(nbr,),
                device_id_type=pl.DeviceIdType.MESH,
            )
        pl.semaphore_wait(barrier_sem, 2)

        out_ref[pl.ds(my * m_per, m_per), :] = x_ref[:, :].astype(
            out_ref.dtype
        )

        sends = []

        def start(idx, piece, s_sems, r_sems, i, dev):
            s = make(idx, piece, s_sems.at[i], r_sems.at[i], dev)
            s.start()
            sends.append(s)

        for p in range(2):
            start(my, p, send_r, recv_r, p, right)
            start(my, p, send_l, recv_l, p, left)

        for h in range(N_HOP):
            cr = chunk_at(h + 1)
            cl = chunk_at(N_DEV - h - 1)
            for p in range(2):
                i = 2 * h + p
                last = N_HOP if p == 0 else N_HOP - 1
                if h < last:
                    make(cr, p, send_r.at[i], recv_r.at[i],
                         left).wait_recv()
                    if h + 1 < last:
                        start(cr, p, send_r, recv_r, i + 2, right)
                last = N_HOP if p == 1 else N_HOP - 1
                if h < last:
                    make(cl, p, send_l.at[i], recv_l.at[i],
                         right).wait_recv()
                    if h + 1 < last:
                        start(cl, p, send_l, recv_l, i + 2, left)

        for s in sends:
            s.wait_send()

    ring_tab = jnp.asarray(_RING)
    pos_tab = jnp.asarray(_POS)
    return pl.pallas_call(
        body,
        out_shape=jax.ShapeDtypeStruct((N_DEV * m_per, n), jnp.bfloat16),
        in_specs=[
            pl.BlockSpec(memory_space=pltpu.SMEM),
            pl.BlockSpec(memory_space=pltpu.SMEM),
            pl.BlockSpec(memory_space=pltpu.VMEM),
        ],
        out_specs=pl.BlockSpec(memory_space=pltpu.VMEM),
        scratch_shapes=[
            pltpu.SemaphoreType.DMA((2 * N_HOP,)),
            pltpu.SemaphoreType.DMA((2 * N_HOP,)),
            pltpu.SemaphoreType.DMA((2 * N_HOP,)),
            pltpu.SemaphoreType.DMA((2 * N_HOP,)),
        ],
        compiler_params=pltpu.CompilerParams(collective_id=0),
    )(ring_tab, pos_tab, x)


# device time: 103735 ns/iter; 1.0182x vs baseline; 1.0007x over previous
import numpy as np

import jax
import jax.numpy as jnp
from jax import lax
from jax.experimental import pallas as pl
from jax.experimental.pallas import tpu as pltpu

N_DEV = 32

_RING = np.array(
    [1, 2, 5, 6, 14, 13, 10, 9, 17, 18, 21, 22, 30, 29, 26, 25,
     24, 27, 28, 31, 23, 20, 19, 16, 8, 11, 12, 15, 7, 4, 3, 0],
    dtype=np.int32,
)
_POS = np.zeros(N_DEV, dtype=np.int32)
_POS[_RING] = np.arange(N_DEV, dtype=np.int32)

N_HOP = N_DEV // 2


def kernel(x):
    m_per, n = x.shape

    def body(ring_ref, pos_ref, x_ref, out_ref,
             send_r, recv_r, send_l, recv_l):
        my = lax.axis_index("i")
        r = pos_ref[my]
        right = ring_ref[(r + 1) % N_DEV]
        left = ring_ref[(r + N_DEV - 1) % N_DEV]

        def chunk_at(back):
            return ring_ref[(r + N_DEV - back) % N_DEV]

        half = m_per // 2

        def make(idx, piece, s_sem, r_sem, dev):
            sl = pl.ds(piece * half, half)
            return pltpu.make_async_remote_copy(
                src_ref=out_ref.at[idx, sl, :],
                dst_ref=out_ref.at[idx, sl, :],
                send_sem=s_sem,
                recv_sem=r_sem,
                device_id=(dev,),
                device_id_type=pl.DeviceIdType.MESH,
            )

        out_ref[my, :, :] = x_ref[:, :].astype(out_ref.dtype)

        barrier_sem = pltpu.get_barrier_semaphore()
        for nbr in (left, right):
            pl.semaphore_signal(
                barrier_sem,
                inc=1,
                device_id=(nbr,),
                device_id_type=pl.DeviceIdType.MESH,
            )
        pl.semaphore_wait(barrier_sem, 2)

        sends = []

        def start(idx, piece, s_sems, r_sems, i, dev):
            s = make(idx, piece, s_sems.at[i], r_sems.at[i], dev)
            s.start()
            sends.append(s)

        for p in range(2):
            start(my, p, send_r, recv_r, p, right)
            start(my, p, send_l, recv_l, p, left)

        for h in range(N_HOP):
            cr = chunk_at(h + 1)
            cl = chunk_at(N_DEV - h - 1)
            for p in range(2):
                i = 2 * h + p
                last = N_HOP if p == 0 else N_HOP - 1
                if h < last:
                    make(cr, p, send_r.at[i], recv_r.at[i],
                         left).wait_recv()
                    if h + 1 < last:
                        start(cr, p, send_r, recv_r, i + 2, right)
                last = N_HOP if p == 1 else N_HOP - 1
                if h < last:
                    make(cl, p, send_l.at[i], recv_l.at[i],
                         right).wait_recv()
                    if h + 1 < last:
                        start(cl, p, send_l, recv_l, i + 2, left)

        for s in sends:
            s.wait_send()

    ring_tab = jnp.asarray(_RING)
    pos_tab = jnp.asarray(_POS)
    out3 = pl.pallas_call(
        body,
        out_shape=jax.ShapeDtypeStruct((N_DEV, m_per, n), jnp.bfloat16),
        in_specs=[
            pl.BlockSpec(memory_space=pltpu.SMEM),
            pl.BlockSpec(memory_space=pltpu.SMEM),
            pl.BlockSpec(memory_space=pltpu.VMEM),
        ],
        out_specs=pl.BlockSpec(memory_space=pltpu.VMEM),
        scratch_shapes=[
            pltpu.SemaphoreType.DMA((2 * N_HOP,)),
            pltpu.SemaphoreType.DMA((2 * N_HOP,)),
            pltpu.SemaphoreType.DMA((2 * N_HOP,)),
            pltpu.SemaphoreType.DMA((2 * N_HOP,)),
        ],
        compiler_params=pltpu.CompilerParams(collective_id=0),
    )(ring_tab, pos_tab, x)
    return jnp.reshape(out3, (N_DEV * m_per, n))


# device time: 103715 ns/iter; 1.0184x vs baseline; 1.0002x over previous
import numpy as np

import jax
import jax.numpy as jnp
from jax import lax
from jax.experimental import pallas as pl
from jax.experimental.pallas import tpu as pltpu

N_DEV = 32

_RING = np.array(
    [1, 2, 5, 6, 14, 13, 10, 9, 17, 18, 21, 22, 30, 29, 26, 25,
     24, 27, 28, 31, 23, 20, 19, 16, 8, 11, 12, 15, 7, 4, 3, 0],
    dtype=np.int32,
)
_POS = np.zeros(N_DEV, dtype=np.int32)
_POS[_RING] = np.arange(N_DEV, dtype=np.int32)

N_HOP = N_DEV // 2


def kernel(x):
    m_per, n = x.shape

    def body(ring_ref, pos_ref, x_ref, out_ref,
             send_r, recv_r, send_l, recv_l):
        my = lax.axis_index("i")
        r = pos_ref[my]
        right = ring_ref[(r + 1) % N_DEV]
        left = ring_ref[(r + N_DEV - 1) % N_DEV]

        def chunk_at(back):
            return ring_ref[(r + N_DEV - back) % N_DEV]

        half = m_per // 2

        def make(idx, piece, s_sem, r_sem, dev):
            sl = pl.ds(idx * m_per + piece * half, half)
            return pltpu.make_async_remote_copy(
                src_ref=out_ref.at[sl, :],
                dst_ref=out_ref.at[sl, :],
                send_sem=s_sem,
                recv_sem=r_sem,
                device_id=(dev,),
                device_id_type=pl.DeviceIdType.MESH,
            )

        out_ref[pl.ds(my * m_per, m_per), :] = x_ref[:, :].astype(
            out_ref.dtype
        )

        barrier_sem = pltpu.get_barrier_semaphore()
        for nbr in (left, right):
            pl.semaphore_signal(
                barrier_sem,
                inc=1,
                device_id=(nbr,),
                device_id_type=pl.DeviceIdType.MESH,
            )
        pl.semaphore_wait(barrier_sem, 2)

        sends = []

        def start(idx, piece, s_sems, r_sems, i, dev):
            s = make(idx, piece, s_sems.at[i], r_sems.at[i], dev)
            s.start()
            sends.append(s)

        for p in range(2):
            start(my, p, send_r, recv_r, p, right)
            start(my, p, send_l, recv_l, p, left)

        for h in range(N_HOP):
            cr = chunk_at(h + 1)
            cl = chunk_at(N_DEV - h - 1)
            for p in range(2):
                i = 2 * h + p
                last = N_HOP if p == 0 else N_HOP - 1
                if h < last:
                    make(cr, p, send_r.at[i], recv_r.at[i],
                         left).wait_recv()
                    if h + 1 < last:
                        start(cr, p, send_r, recv_r, i + 2, right)
                last = N_HOP if p == 1 else N_HOP - 1
                if h < last:
                    make(cl, p, send_l.at[i], recv_l.at[i],
                         right).wait_recv()
                    if h + 1 < last:
                        start(cl, p, send_l, recv_l, i + 2, left)

        for s in sends:
            s.wait_send()

    ring_tab = jnp.asarray(_RING)
    pos_tab = jnp.asarray(_POS)
    return pl.pallas_call(
        body,
        out_shape=jax.ShapeDtypeStruct((N_DEV * m_per, n), jnp.bfloat16),
        in_specs=[
            pl.BlockSpec(memory_space=pltpu.SMEM),
            pl.BlockSpec(memory_space=pltpu.SMEM),
            pl.BlockSpec(memory_space=pltpu.VMEM),
        ],
        out_specs=pl.BlockSpec(memory_space=pltpu.VMEM),
        scratch_shapes=[
            pltpu.SemaphoreType.DMA((2 * N_HOP,)),
            pltpu.SemaphoreType.DMA((2 * N_HOP,)),
            pltpu.SemaphoreType.DMA((2 * N_HOP,)),
            pltpu.SemaphoreType.DMA((2 * N_HOP,)),
        ],
        compiler_params=pltpu.CompilerParams(collective_id=0),
    )(ring_tab, pos_tab, x)
